# baseline (speedup 1.0000x reference)
import functools

import jax
import jax.numpy as jnp
from jax import lax
from jax.experimental import pallas as pl
from jax.experimental.pallas import tpu as pltpu
from jax.experimental.pallas import tpu_sc as plsc

B = 4096
L = 200
EMB = 64
HID = 128
V = 1000000

NC = 2
NS = 16
NW = NC * NS
BPW = B // NW
GATHER = L
STEPS = BPW
NBUF = 4
ROWS_PER_ACC = 10

NCHUNK = V // 128
RPW = NCHUNK // NW
RTAIL = V - NCHUNK * 128


def _repack_build():
    mesh = plsc.VectorSubcoreMesh(core_axis_name="c", subcore_axis_name="s")

    @functools.partial(
        pl.kernel,
        out_type=jax.ShapeDtypeStruct((V // 2, 128), jnp.float32),
        mesh=mesh,
        compiler_params=pltpu.CompilerParams(needs_layout_passes=False),
        scratch_types=[
            pltpu.VMEM((2, 64, 128), jnp.float32),
            pltpu.VMEM((2, 64, 128), jnp.float32),
            pltpu.VMEM((64, 64), jnp.float32),
            pltpu.VMEM((32, 128), jnp.float32),
            pltpu.SemaphoreType.DMA,
            pltpu.SemaphoreType.DMA,
            pltpu.SemaphoreType.DMA,
            pltpu.SemaphoreType.DMA,
        ],
    )
    def repack(embt_hbm, tail_hbm, out_hbm, in_v, out_v, tin_v, tout_v,
               i0, i1, o0, o1):
        isems = [i0, i1]
        osems = [o0, o1]
        wid = lax.axis_index("s") * NC + lax.axis_index("c")
        base = wid * RPW

        iota = lax.iota(jnp.int32, 16)

        def issue_in(c, s):
            pltpu.async_copy(
                embt_hbm.at[:, pl.ds(c * 128, 128)], in_v.at[s], isems[s]
            )

        def wait_in(s):
            pltpu.make_async_copy(
                embt_hbm.at[:, pl.ds(0, 128)], in_v.at[s], isems[s]
            ).wait()

        def issue_out(c, s):
            pltpu.async_copy(
                out_v.at[s], out_hbm.at[pl.ds(c * 64, 64)], osems[s]
            )

        def wait_out(s):
            pltpu.make_async_copy(
                out_v.at[s], out_hbm.at[pl.ds(0, 64)], osems[s]
            ).wait()

        def transpose_block(src, dst, npairs):
            def row(p, carry):
                for k in range(8):
                    col = jnp.full((16,), 2 * p + k // 4, jnp.int32)
                    seg = plsc.load_gather(src, [iota + 16 * (k % 4), col])
                    dst[p, pl.ds(16 * k, 16)] = seg
                return carry
            lax.fori_loop(0, npairs, row, 0)

        for s in range(2):
            issue_out(base + s, s)
            issue_in(base + s, s)

        def rstep(i, issue_next):
            for s in range(2):
                c = base + i * 2 + s
                wait_in(s)
                wait_out(s)
                transpose_block(in_v.at[s], out_v.at[s], 64)
                issue_out(c, s)
                if issue_next:
                    issue_in(c + 2, s)

        lax.fori_loop(0, RPW // 2 - 1, lambda i, cr: (rstep(i, True), cr)[1], 0)
        rstep(RPW // 2 - 1, False)
        wait_out(0)
        wait_out(1)

        @pl.when(wid < NCHUNK - NW * RPW)
        def _():
            c = NW * RPW + wid
            pltpu.sync_copy(embt_hbm.at[:, pl.ds(c * 128, 128)], in_v.at[0])
            transpose_block(in_v.at[0], out_v.at[0], 64)
            pltpu.sync_copy(out_v.at[0], out_hbm.at[pl.ds(c * 64, 64)])

        @pl.when(wid == NW - 1)
        def _():
            pltpu.sync_copy(tail_hbm, tin_v)
            def row(p, carry):
                for k in range(8):
                    col = jnp.full((16,), 2 * p + k // 4, jnp.int32)
                    seg = plsc.load_gather(tin_v, [iota + 16 * (k % 4), col])
                    tout_v[p, pl.ds(16 * k, 16)] = seg
                return carry
            lax.fori_loop(0, 32, row, 0)
            pltpu.sync_copy(
                tout_v, out_hbm.at[pl.ds((V - RTAIL) // 2, RTAIL // 2)]
            )

    return repack


def _cbow_pool_build():
    mesh = plsc.VectorSubcoreMesh(core_axis_name="c", subcore_axis_name="s")

    @functools.partial(
        pl.kernel,
        out_type=jax.ShapeDtypeStruct((B, EMB), jnp.float32),
        mesh=mesh,
        compiler_params=pltpu.CompilerParams(use_tc_tiling_on_sc=False),
        scratch_types=[
            pltpu.VMEM((BPW, 128), jnp.int32),
            pltpu.VMEM((BPW, 128), jnp.int32),
            pltpu.VMEM((NBUF, GATHER, EMB), jnp.float32),
            pltpu.VMEM((BPW, EMB), jnp.float32),
            pltpu.SemaphoreType.DMA,
            pltpu.SemaphoreType.DMA,
            pltpu.SemaphoreType.DMA,
            pltpu.SemaphoreType.DMA,
        ],
    )
    def pool(wa_hbm, wb_hbm, emb_hbm, out_hbm, ia_v, ib_v, rows_v, hout_v,
             s0, s1, s2, s3):
        sems = [s0, s1, s2, s3]
        wid = lax.axis_index("s") * NC + lax.axis_index("c")

        pltpu.sync_copy(wa_hbm.at[pl.ds(wid * BPW, BPW)], ia_v)
        pltpu.sync_copy(wb_hbm.at[pl.ds(wid * BPW, BPW)], ib_v)

        def issue(b, s):
            pltpu.async_copy(
                emb_hbm.at[ia_v.at[b]], rows_v.at[s, pl.ds(0, 128)], sems[s]
            )
            pltpu.async_copy(
                emb_hbm.at[ib_v.at[b, pl.ds(56, 72)]],
                rows_v.at[s, pl.ds(128, 72)],
                sems[s],
            )

        for s in range(NBUF):
            issue(s, s)

        def accum(s):
            def body(r10, acc):
                a0, a1, a2, a3 = acc
                for u in range(ROWS_PER_ACC):
                    r = r10 * ROWS_PER_ACC + u
                    a0 = a0 + rows_v[s, r, pl.ds(0, 16)]
                    a1 = a1 + rows_v[s, r, pl.ds(16, 16)]
                    a2 = a2 + rows_v[s, r, pl.ds(32, 16)]
                    a3 = a3 + rows_v[s, r, pl.ds(48, 16)]
                return (a0, a1, a2, a3)
            zeros4 = tuple(jnp.zeros((16,), jnp.float32) for _ in range(4))
            return lax.fori_loop(0, GATHER // ROWS_PER_ACC, body, zeros4)

        def wait(s):
            pltpu.make_async_copy(
                emb_hbm.at[ia_v.at[0]], rows_v.at[s, pl.ds(0, 128)], sems[s]
            ).wait()
            pltpu.make_async_copy(
                emb_hbm.at[ib_v.at[0, pl.ds(56, 72)]],
                rows_v.at[s, pl.ds(128, 72)],
                sems[s],
            ).wait()

        def step_block(i, issue_next):
            for s in range(NBUF):
                wait(s)
                acc = accum(s)
                b = i * NBUF + s
                hout_v[b, pl.ds(0, 16)] = acc[0]
                hout_v[b, pl.ds(16, 16)] = acc[1]
                hout_v[b, pl.ds(32, 16)] = acc[2]
                hout_v[b, pl.ds(48, 16)] = acc[3]
                if issue_next:
                    issue(b + NBUF, s)

        def main_body(i, carry):
            step_block(i, True)
            return carry

        lax.fori_loop(0, STEPS // NBUF - 1, main_body, 0)
        step_block(STEPS // NBUF - 1, False)

        pltpu.sync_copy(hout_v, out_hbm.at[pl.ds(wid * BPW, BPW)])

    return pool


_repack = _repack_build()
_cbow_pool = _cbow_pool_build()


def _mlp_body(h_ref, w0_ref, b0_ref, w1_ref, b1_ref, wout_ref, bout_ref, o_ref):
    h = h_ref[...]
    t = jnp.tanh(jnp.dot(h, w0_ref[...], preferred_element_type=jnp.float32)
                 + b0_ref[...])
    t = jnp.tanh(jnp.dot(t, w1_ref[...], preferred_element_type=jnp.float32)
                 + b1_ref[...])
    o_ref[...] = (jnp.sum(t * wout_ref[...], axis=1, keepdims=True)
                  + bout_ref[...])


def kernel(words, emb, W0, b0, W1, b1, Wout, bout):
    w32 = words.astype(jnp.int32)
    wa = w32[:, :128]
    wb = w32[:, L - 128:]
    embt = emb.T
    tail = emb[NCHUNK * 128:].T
    packed = _repack(embt, tail)
    emb_lin = packed.reshape(V, EMB)
    h = _cbow_pool(wa, wb, emb_lin)
    out = pl.pallas_call(
        _mlp_body,
        out_shape=jax.ShapeDtypeStruct((B, 1), jnp.float32),
    )(
        h,
        W0,
        b0.reshape(1, HID),
        W1,
        b1.reshape(1, HID),
        Wout.reshape(1, HID),
        bout.reshape(1, 1),
    )
    return out


# speedup vs baseline: 1.2109x; 1.2109x over previous
import functools

import jax
import jax.numpy as jnp
from jax import lax
from jax.experimental import pallas as pl
from jax.experimental.pallas import tpu as pltpu
from jax.experimental.pallas import tpu_sc as plsc

B = 4096
L = 200
EMB = 64
HID = 128
V = 1000000

NC = 2
NS = 16
NW = NC * NS
BPW = B // NW
GATHER = L
STEPS = BPW
NBUF = 4
ROWS_PER_ACC = 10

NCHUNK = V // 128
RPW = NCHUNK // NW
RTAIL = V - NCHUNK * 128


RP_W = 512
RP_GRID = -(-V // RP_W)


def _repack_tc_body(in_ref, out_ref):
    x = in_ref[...]
    idx = jax.lax.broadcasted_iota(jnp.int32, (EMB, 64), 1) * 2
    for kb in range(RP_W // 128):
        xs = x[:, kb * 128:(kb + 1) * 128]
        x_even = jnp.take_along_axis(xs, idx, axis=1)
        x_odd = jnp.take_along_axis(xs, idx + 1, axis=1)
        m = jnp.concatenate([x_even, x_odd], axis=0)
        out_ref[pl.ds(kb * 64, 64), :] = m.T


def _repack(embt):
    return pl.pallas_call(
        _repack_tc_body,
        grid=(RP_GRID,),
        in_specs=[pl.BlockSpec((EMB, RP_W), lambda j: (0, j))],
        out_specs=pl.BlockSpec((RP_W // 2, 128), lambda j: (j, 0)),
        out_shape=jax.ShapeDtypeStruct((V // 2, 128), jnp.float32),
    )(embt)


def _cbow_pool_build():
    mesh = plsc.VectorSubcoreMesh(core_axis_name="c", subcore_axis_name="s")

    @functools.partial(
        pl.kernel,
        out_type=jax.ShapeDtypeStruct((B, EMB), jnp.float32),
        mesh=mesh,
        compiler_params=pltpu.CompilerParams(use_tc_tiling_on_sc=False),
        scratch_types=[
            pltpu.VMEM((BPW, 128), jnp.int32),
            pltpu.VMEM((BPW, 128), jnp.int32),
            pltpu.VMEM((NBUF, GATHER, EMB), jnp.float32),
            pltpu.VMEM((BPW, EMB), jnp.float32),
            pltpu.SemaphoreType.DMA,
            pltpu.SemaphoreType.DMA,
            pltpu.SemaphoreType.DMA,
            pltpu.SemaphoreType.DMA,
        ],
    )
    def pool(wa_hbm, wb_hbm, emb_hbm, out_hbm, ia_v, ib_v, rows_v, hout_v,
             s0, s1, s2, s3):
        sems = [s0, s1, s2, s3]
        wid = lax.axis_index("s") * NC + lax.axis_index("c")

        pltpu.sync_copy(wa_hbm.at[pl.ds(wid * BPW, BPW)], ia_v)
        pltpu.sync_copy(wb_hbm.at[pl.ds(wid * BPW, BPW)], ib_v)

        def issue(b, s):
            pltpu.async_copy(
                emb_hbm.at[ia_v.at[b]], rows_v.at[s, pl.ds(0, 128)], sems[s]
            )
            pltpu.async_copy(
                emb_hbm.at[ib_v.at[b, pl.ds(56, 72)]],
                rows_v.at[s, pl.ds(128, 72)],
                sems[s],
            )

        for s in range(NBUF):
            issue(s, s)

        def accum(s):
            def body(r10, acc):
                a0, a1, a2, a3 = acc
                for u in range(ROWS_PER_ACC):
                    r = r10 * ROWS_PER_ACC + u
                    a0 = a0 + rows_v[s, r, pl.ds(0, 16)]
                    a1 = a1 + rows_v[s, r, pl.ds(16, 16)]
                    a2 = a2 + rows_v[s, r, pl.ds(32, 16)]
                    a3 = a3 + rows_v[s, r, pl.ds(48, 16)]
                return (a0, a1, a2, a3)
            zeros4 = tuple(jnp.zeros((16,), jnp.float32) for _ in range(4))
            return lax.fori_loop(0, GATHER // ROWS_PER_ACC, body, zeros4)

        def wait(s):
            pltpu.make_async_copy(
                emb_hbm.at[ia_v.at[0]], rows_v.at[s, pl.ds(0, 128)], sems[s]
            ).wait()
            pltpu.make_async_copy(
                emb_hbm.at[ib_v.at[0, pl.ds(56, 72)]],
                rows_v.at[s, pl.ds(128, 72)],
                sems[s],
            ).wait()

        def step_block(i, issue_next):
            for s in range(NBUF):
                wait(s)
                acc = accum(s)
                b = i * NBUF + s
                hout_v[b, pl.ds(0, 16)] = acc[0]
                hout_v[b, pl.ds(16, 16)] = acc[1]
                hout_v[b, pl.ds(32, 16)] = acc[2]
                hout_v[b, pl.ds(48, 16)] = acc[3]
                if issue_next:
                    issue(b + NBUF, s)

        def main_body(i, carry):
            step_block(i, True)
            return carry

        lax.fori_loop(0, STEPS // NBUF - 1, main_body, 0)
        step_block(STEPS // NBUF - 1, False)

        pltpu.sync_copy(hout_v, out_hbm.at[pl.ds(wid * BPW, BPW)])

    return pool


_cbow_pool = _cbow_pool_build()


def _mlp_body(h_ref, w0_ref, b0_ref, w1_ref, b1_ref, wout_ref, bout_ref, o_ref):
    h = h_ref[...]
    t = jnp.tanh(jnp.dot(h, w0_ref[...], preferred_element_type=jnp.float32)
                 + b0_ref[...])
    t = jnp.tanh(jnp.dot(t, w1_ref[...], preferred_element_type=jnp.float32)
                 + b1_ref[...])
    o_ref[...] = (jnp.sum(t * wout_ref[...], axis=1, keepdims=True)
                  + bout_ref[...])


def kernel(words, emb, W0, b0, W1, b1, Wout, bout):
    w32 = words.astype(jnp.int32)
    wa = w32[:, :128]
    wb = w32[:, L - 128:]
    packed = _repack(emb.T)
    emb_lin = packed.reshape(V, EMB)
    h = _cbow_pool(wa, wb, emb_lin)
    out = pl.pallas_call(
        _mlp_body,
        out_shape=jax.ShapeDtypeStruct((B, 1), jnp.float32),
    )(
        h,
        W0,
        b0.reshape(1, HID),
        W1,
        b1.reshape(1, HID),
        Wout.reshape(1, HID),
        bout.reshape(1, 1),
    )
    return out


# speedup vs baseline: 1.2184x; 1.0062x over previous
import functools

import jax
import jax.numpy as jnp
from jax import lax
from jax.experimental import pallas as pl
from jax.experimental.pallas import tpu as pltpu
from jax.experimental.pallas import tpu_sc as plsc

B = 4096
L = 200
EMB = 64
HID = 128
V = 1000000

NC = 2
NS = 16
NW = NC * NS
BPW = B // NW
GATHER = L
STEPS = BPW
NBUF = 4
ROWS_PER_ACC = 10

NCHUNK = V // 128
RPW = NCHUNK // NW
RTAIL = V - NCHUNK * 128


def _repack_build():
    mesh = plsc.VectorSubcoreMesh(core_axis_name="c", subcore_axis_name="s")

    @functools.partial(
        pl.kernel,
        out_type=jax.ShapeDtypeStruct((V // 2, 128), jnp.float32),
        mesh=mesh,
        compiler_params=pltpu.CompilerParams(needs_layout_passes=False),
        scratch_types=[
            pltpu.VMEM((2, 64, 128), jnp.float32),
            pltpu.VMEM((2, 64, 128), jnp.float32),
            pltpu.VMEM((64, 64), jnp.float32),
            pltpu.VMEM((32, 128), jnp.float32),
            pltpu.SemaphoreType.DMA,
            pltpu.SemaphoreType.DMA,
            pltpu.SemaphoreType.DMA,
            pltpu.SemaphoreType.DMA,
        ],
    )
    def repack(embt_hbm, tail_hbm, out_hbm, in_v, out_v, tin_v, tout_v,
               i0, i1, o0, o1):
        isems = [i0, i1]
        osems = [o0, o1]
        wid = lax.axis_index("s") * NC + lax.axis_index("c")
        base = wid * RPW

        iota = lax.iota(jnp.int32, 16)
        pidx = [16 * j // 2 + lax.shift_right_logical(iota, 1)
                for j in range(8)]
        cbase = lax.mul(lax.rem(iota, 2), jnp.full((16,), 64, jnp.int32))

        def issue_in(c, s):
            pltpu.async_copy(
                embt_hbm.at[:, pl.ds(c * 128, 128)], in_v.at[s], isems[s]
            )

        def wait_in(s):
            pltpu.make_async_copy(
                embt_hbm.at[:, pl.ds(0, 128)], in_v.at[s], isems[s]
            ).wait()

        def issue_out(c, s):
            pltpu.async_copy(
                out_v.at[s], out_hbm.at[pl.ds(c * 64, 64)], osems[s]
            )

        def wait_out(s):
            pltpu.make_async_copy(
                out_v.at[s], out_hbm.at[pl.ds(0, 64)], osems[s]
            ).wait()

        def transpose_rows(src, dst, d0, nrows):
            def row(d, carry):
                cidx = cbase + d
                for j in range(8):
                    val = src[d, pl.ds(16 * j, 16)]
                    plsc.store_scatter(dst, [pidx[j], cidx], val)
                return carry
            lax.fori_loop(d0, d0 + nrows, row, 0, unroll=4)

        for s in range(2):
            issue_out(base + s, s)
            issue_in(base + s, s)

        def rstep(i, issue_next):
            for s in range(2):
                c = base + i * 2 + s
                wait_in(s)
                wait_out(s)
                transpose_rows(in_v.at[s], out_v.at[s], 0, 64)
                issue_out(c, s)
                if issue_next:
                    issue_in(c + 2, s)

        lax.fori_loop(0, RPW // 2 - 1, lambda i, cr: (rstep(i, True), cr)[1], 0)
        rstep(RPW // 2 - 1, False)
        wait_out(0)
        wait_out(1)

        @pl.when(wid < NCHUNK - NW * RPW)
        def _():
            c = NW * RPW + wid
            pltpu.sync_copy(embt_hbm.at[:, pl.ds(c * 128, 128)], in_v.at[0])
            transpose_rows(in_v.at[0], out_v.at[0], 0, 64)
            pltpu.sync_copy(out_v.at[0], out_hbm.at[pl.ds(c * 64, 64)])

        @pl.when(wid == NW - 1)
        def _():
            pltpu.sync_copy(tail_hbm, tin_v)
            def row(d, carry):
                cidx = cbase + d
                for j in range(4):
                    val = tin_v[d, pl.ds(16 * j, 16)]
                    plsc.store_scatter(tout_v, [pidx[j], cidx], val)
                return carry
            lax.fori_loop(0, 64, row, 0, unroll=4)
            pltpu.sync_copy(
                tout_v, out_hbm.at[pl.ds((V - RTAIL) // 2, RTAIL // 2)]
            )

    return repack


_repack = _repack_build()


def _cbow_pool_build():
    mesh = plsc.VectorSubcoreMesh(core_axis_name="c", subcore_axis_name="s")

    @functools.partial(
        pl.kernel,
        out_type=jax.ShapeDtypeStruct((B, EMB), jnp.float32),
        mesh=mesh,
        compiler_params=pltpu.CompilerParams(use_tc_tiling_on_sc=False),
        scratch_types=[
            pltpu.VMEM((BPW, 128), jnp.int32),
            pltpu.VMEM((BPW, 128), jnp.int32),
            pltpu.VMEM((NBUF, GATHER, EMB), jnp.float32),
            pltpu.VMEM((BPW, EMB), jnp.float32),
            pltpu.SemaphoreType.DMA,
            pltpu.SemaphoreType.DMA,
            pltpu.SemaphoreType.DMA,
            pltpu.SemaphoreType.DMA,
        ],
    )
    def pool(wa_hbm, wb_hbm, emb_hbm, out_hbm, ia_v, ib_v, rows_v, hout_v,
             s0, s1, s2, s3):
        sems = [s0, s1, s2, s3]
        wid = lax.axis_index("s") * NC + lax.axis_index("c")

        pltpu.sync_copy(wa_hbm.at[pl.ds(wid * BPW, BPW)], ia_v)
        pltpu.sync_copy(wb_hbm.at[pl.ds(wid * BPW, BPW)], ib_v)

        def issue(b, s):
            pltpu.async_copy(
                emb_hbm.at[ia_v.at[b]], rows_v.at[s, pl.ds(0, 128)], sems[s]
            )
            pltpu.async_copy(
                emb_hbm.at[ib_v.at[b, pl.ds(56, 72)]],
                rows_v.at[s, pl.ds(128, 72)],
                sems[s],
            )

        for s in range(NBUF):
            issue(s, s)

        def accum(s):
            def body(r10, acc):
                a0, a1, a2, a3 = acc
                for u in range(ROWS_PER_ACC):
                    r = r10 * ROWS_PER_ACC + u
                    a0 = a0 + rows_v[s, r, pl.ds(0, 16)]
                    a1 = a1 + rows_v[s, r, pl.ds(16, 16)]
                    a2 = a2 + rows_v[s, r, pl.ds(32, 16)]
                    a3 = a3 + rows_v[s, r, pl.ds(48, 16)]
                return (a0, a1, a2, a3)
            zeros4 = tuple(jnp.zeros((16,), jnp.float32) for _ in range(4))
            return lax.fori_loop(0, GATHER // ROWS_PER_ACC, body, zeros4)

        def wait(s):
            pltpu.make_async_copy(
                emb_hbm.at[ia_v.at[0]], rows_v.at[s, pl.ds(0, 128)], sems[s]
            ).wait()
            pltpu.make_async_copy(
                emb_hbm.at[ib_v.at[0, pl.ds(56, 72)]],
                rows_v.at[s, pl.ds(128, 72)],
                sems[s],
            ).wait()

        def step_block(i, issue_next):
            for s in range(NBUF):
                wait(s)
                acc = accum(s)
                b = i * NBUF + s
                hout_v[b, pl.ds(0, 16)] = acc[0]
                hout_v[b, pl.ds(16, 16)] = acc[1]
                hout_v[b, pl.ds(32, 16)] = acc[2]
                hout_v[b, pl.ds(48, 16)] = acc[3]
                if issue_next:
                    issue(b + NBUF, s)

        def main_body(i, carry):
            step_block(i, True)
            return carry

        lax.fori_loop(0, STEPS // NBUF - 1, main_body, 0)
        step_block(STEPS // NBUF - 1, False)

        pltpu.sync_copy(hout_v, out_hbm.at[pl.ds(wid * BPW, BPW)])

    return pool


_cbow_pool = _cbow_pool_build()


def _mlp_body(h_ref, w0_ref, b0_ref, w1_ref, b1_ref, wout_ref, bout_ref, o_ref):
    h = h_ref[...]
    t = jnp.tanh(jnp.dot(h, w0_ref[...], preferred_element_type=jnp.float32)
                 + b0_ref[...])
    t = jnp.tanh(jnp.dot(t, w1_ref[...], preferred_element_type=jnp.float32)
                 + b1_ref[...])
    o_ref[...] = (jnp.sum(t * wout_ref[...], axis=1, keepdims=True)
                  + bout_ref[...])


def kernel(words, emb, W0, b0, W1, b1, Wout, bout):
    w32 = words.astype(jnp.int32)
    wa = w32[:, :128]
    wb = w32[:, L - 128:]
    packed = _repack(emb.T, emb[NCHUNK * 128:].T)
    emb_lin = packed.reshape(V, EMB)
    h = _cbow_pool(wa, wb, emb_lin)
    out = pl.pallas_call(
        _mlp_body,
        out_shape=jax.ShapeDtypeStruct((B, 1), jnp.float32),
    )(
        h,
        W0,
        b0.reshape(1, HID),
        W1,
        b1.reshape(1, HID),
        Wout.reshape(1, HID),
        bout.reshape(1, 1),
    )
    return out


# speedup vs baseline: 1.6620x; 1.3641x over previous
import functools

import jax
import jax.numpy as jnp
from jax import lax
from jax.experimental import pallas as pl
from jax.experimental.pallas import tpu as pltpu
from jax.experimental.pallas import tpu_sc as plsc

B = 4096
L = 200
EMB = 64
HID = 128
V = 1000000

NC = 2
NS = 16
NW = NC * NS
BPW = B // NW
GATHER = L
STEPS = BPW
NBUF = 4
ROWS_PER_ACC = 10

NCHUNK = V // 128
RPW = NCHUNK // NW
RTAIL = V - NCHUNK * 128


def _repack_build():
    mesh = plsc.VectorSubcoreMesh(core_axis_name="c", subcore_axis_name="s")

    @functools.partial(
        pl.kernel,
        out_type=jax.ShapeDtypeStruct((V // 2, 128), jnp.float32),
        mesh=mesh,
        compiler_params=pltpu.CompilerParams(needs_layout_passes=False),
        scratch_types=[
            pltpu.VMEM((2, 64, 128), jnp.float32),
            pltpu.VMEM((2, 64, 128), jnp.float32),
            pltpu.VMEM((64, 64), jnp.float32),
            pltpu.VMEM((32, 128), jnp.float32),
            pltpu.SemaphoreType.DMA,
            pltpu.SemaphoreType.DMA,
            pltpu.SemaphoreType.DMA,
            pltpu.SemaphoreType.DMA,
        ],
    )
    def repack(embt_hbm, tail_hbm, out_hbm, in_v, out_v, tin_v, tout_v,
               i0, i1, o0, o1):
        isems = [i0, i1]
        osems = [o0, o1]
        wid = lax.axis_index("s") * NC + lax.axis_index("c")
        base = wid * RPW

        iota = lax.iota(jnp.int32, 16)
        pidx = [16 * j // 2 + lax.shift_right_logical(iota, 1)
                for j in range(8)]
        cbase = lax.mul(lax.rem(iota, 2), jnp.full((16,), 64, jnp.int32))

        def issue_in(c, s):
            pltpu.async_copy(
                embt_hbm.at[:, pl.ds(c * 128, 128)], in_v.at[s], isems[s]
            )

        def wait_in(s):
            pltpu.make_async_copy(
                embt_hbm.at[:, pl.ds(0, 128)], in_v.at[s], isems[s]
            ).wait()

        def issue_out(c, s):
            pltpu.async_copy(
                out_v.at[s], out_hbm.at[pl.ds(c * 64, 64)], osems[s]
            )

        def wait_out(s):
            pltpu.make_async_copy(
                out_v.at[s], out_hbm.at[pl.ds(0, 64)], osems[s]
            ).wait()

        def transpose_rows(src, dst, d0, nrows):
            @plsc.parallel_loop(d0, d0 + nrows, step=1, unroll=4)
            def _row(d):
                cidx = cbase + d
                for j in range(8):
                    val = src[d, pl.ds(16 * j, 16)]
                    plsc.store_scatter(dst, [pidx[j], cidx], val)

        for s in range(2):
            issue_out(base + s, s)
            issue_in(base + s, s)

        def rstep(i, issue_next):
            for s in range(2):
                c = base + i * 2 + s
                wait_in(s)
                wait_out(s)
                transpose_rows(in_v.at[s], out_v.at[s], 0, 64)
                issue_out(c, s)
                if issue_next:
                    issue_in(c + 2, s)

        lax.fori_loop(0, RPW // 2 - 1, lambda i, cr: (rstep(i, True), cr)[1], 0)
        rstep(RPW // 2 - 1, False)
        wait_out(0)
        wait_out(1)

        @pl.when(wid < NCHUNK - NW * RPW)
        def _():
            c = NW * RPW + wid
            pltpu.sync_copy(embt_hbm.at[:, pl.ds(c * 128, 128)], in_v.at[0])
            transpose_rows(in_v.at[0], out_v.at[0], 0, 64)
            pltpu.sync_copy(out_v.at[0], out_hbm.at[pl.ds(c * 64, 64)])

        @pl.when(wid == NW - 1)
        def _():
            pltpu.sync_copy(tail_hbm, tin_v)

            @plsc.parallel_loop(0, 64, step=1, unroll=4)
            def _trow(d):
                cidx = cbase + d
                for j in range(4):
                    val = tin_v[d, pl.ds(16 * j, 16)]
                    plsc.store_scatter(tout_v, [pidx[j], cidx], val)
            pltpu.sync_copy(
                tout_v, out_hbm.at[pl.ds((V - RTAIL) // 2, RTAIL // 2)]
            )

    return repack


_repack = _repack_build()


def _cbow_pool_build():
    mesh = plsc.VectorSubcoreMesh(core_axis_name="c", subcore_axis_name="s")

    @functools.partial(
        pl.kernel,
        out_type=jax.ShapeDtypeStruct((B, EMB), jnp.float32),
        mesh=mesh,
        compiler_params=pltpu.CompilerParams(use_tc_tiling_on_sc=False),
        scratch_types=[
            pltpu.VMEM((BPW, 128), jnp.int32),
            pltpu.VMEM((BPW, 128), jnp.int32),
            pltpu.VMEM((NBUF, GATHER, EMB), jnp.float32),
            pltpu.VMEM((BPW, EMB), jnp.float32),
            pltpu.SemaphoreType.DMA,
            pltpu.SemaphoreType.DMA,
            pltpu.SemaphoreType.DMA,
            pltpu.SemaphoreType.DMA,
        ],
    )
    def pool(wa_hbm, wb_hbm, emb_hbm, out_hbm, ia_v, ib_v, rows_v, hout_v,
             s0, s1, s2, s3):
        sems = [s0, s1, s2, s3]
        wid = lax.axis_index("s") * NC + lax.axis_index("c")

        pltpu.sync_copy(wa_hbm.at[pl.ds(wid * BPW, BPW)], ia_v)
        pltpu.sync_copy(wb_hbm.at[pl.ds(wid * BPW, BPW)], ib_v)

        def issue(b, s):
            pltpu.async_copy(
                emb_hbm.at[ia_v.at[b]], rows_v.at[s, pl.ds(0, 128)], sems[s]
            )
            pltpu.async_copy(
                emb_hbm.at[ib_v.at[b, pl.ds(56, 72)]],
                rows_v.at[s, pl.ds(128, 72)],
                sems[s],
            )

        for s in range(NBUF):
            issue(s, s)

        def accum(s):
            def body(r10, acc):
                a0, a1, a2, a3 = acc
                for u in range(ROWS_PER_ACC):
                    r = r10 * ROWS_PER_ACC + u
                    a0 = a0 + rows_v[s, r, pl.ds(0, 16)]
                    a1 = a1 + rows_v[s, r, pl.ds(16, 16)]
                    a2 = a2 + rows_v[s, r, pl.ds(32, 16)]
                    a3 = a3 + rows_v[s, r, pl.ds(48, 16)]
                return (a0, a1, a2, a3)
            zeros4 = tuple(jnp.zeros((16,), jnp.float32) for _ in range(4))
            return lax.fori_loop(0, GATHER // ROWS_PER_ACC, body, zeros4)

        def wait(s):
            pltpu.make_async_copy(
                emb_hbm.at[ia_v.at[0]], rows_v.at[s, pl.ds(0, 128)], sems[s]
            ).wait()
            pltpu.make_async_copy(
                emb_hbm.at[ib_v.at[0, pl.ds(56, 72)]],
                rows_v.at[s, pl.ds(128, 72)],
                sems[s],
            ).wait()

        def step_block(i, issue_next):
            for s in range(NBUF):
                wait(s)
                acc = accum(s)
                b = i * NBUF + s
                hout_v[b, pl.ds(0, 16)] = acc[0]
                hout_v[b, pl.ds(16, 16)] = acc[1]
                hout_v[b, pl.ds(32, 16)] = acc[2]
                hout_v[b, pl.ds(48, 16)] = acc[3]
                if issue_next:
                    issue(b + NBUF, s)

        def main_body(i, carry):
            step_block(i, True)
            return carry

        lax.fori_loop(0, STEPS // NBUF - 1, main_body, 0)
        step_block(STEPS // NBUF - 1, False)

        pltpu.sync_copy(hout_v, out_hbm.at[pl.ds(wid * BPW, BPW)])

    return pool


_cbow_pool = _cbow_pool_build()


def _mlp_body(h_ref, w0_ref, b0_ref, w1_ref, b1_ref, wout_ref, bout_ref, o_ref):
    h = h_ref[...]
    t = jnp.tanh(jnp.dot(h, w0_ref[...], preferred_element_type=jnp.float32)
                 + b0_ref[...])
    t = jnp.tanh(jnp.dot(t, w1_ref[...], preferred_element_type=jnp.float32)
                 + b1_ref[...])
    o_ref[...] = (jnp.sum(t * wout_ref[...], axis=1, keepdims=True)
                  + bout_ref[...])


def kernel(words, emb, W0, b0, W1, b1, Wout, bout):
    w32 = words.astype(jnp.int32)
    wa = w32[:, :128]
    wb = w32[:, L - 128:]
    packed = _repack(emb.T, emb[NCHUNK * 128:].T)
    emb_lin = packed.reshape(V, EMB)
    h = _cbow_pool(wa, wb, emb_lin)
    out = pl.pallas_call(
        _mlp_body,
        out_shape=jax.ShapeDtypeStruct((B, 1), jnp.float32),
    )(
        h,
        W0,
        b0.reshape(1, HID),
        W1,
        b1.reshape(1, HID),
        Wout.reshape(1, HID),
        bout.reshape(1, 1),
    )
    return out


# speedup vs baseline: 1.6630x; 1.0007x over previous
import functools

import jax
import jax.numpy as jnp
from jax import lax
from jax.experimental import pallas as pl
from jax.experimental.pallas import tpu as pltpu
from jax.experimental.pallas import tpu_sc as plsc

B = 4096
L = 200
EMB = 64
HID = 128
V = 1000000

NC = 2
NS = 16
NW = NC * NS
BPW = B // NW
GATHER = L
STEPS = BPW
NBUF = 4
ROWS_PER_ACC = 10

NCHUNK = V // 128
RPW = NCHUNK // NW
RTAIL = V - NCHUNK * 128


def _repack_build():
    mesh = plsc.VectorSubcoreMesh(core_axis_name="c", subcore_axis_name="s")

    @functools.partial(
        pl.kernel,
        out_type=jax.ShapeDtypeStruct((V // 2, 128), jnp.float32),
        mesh=mesh,
        compiler_params=pltpu.CompilerParams(needs_layout_passes=False),
        scratch_types=[
            pltpu.VMEM((2, 64, 128), jnp.float32),
            pltpu.VMEM((2, 64, 128), jnp.float32),
            pltpu.VMEM((64, 64), jnp.float32),
            pltpu.VMEM((32, 128), jnp.float32),
            pltpu.SemaphoreType.DMA,
            pltpu.SemaphoreType.DMA,
            pltpu.SemaphoreType.DMA,
            pltpu.SemaphoreType.DMA,
        ],
    )
    def repack(embt_hbm, tail_hbm, out_hbm, in_v, out_v, tin_v, tout_v,
               i0, i1, o0, o1):
        isems = [i0, i1]
        osems = [o0, o1]
        wid = lax.axis_index("s") * NC + lax.axis_index("c")
        base = wid * RPW

        iota = lax.iota(jnp.int32, 16)
        pidx = [16 * j // 2 + lax.shift_right_logical(iota, 1)
                for j in range(8)]
        cbase = lax.mul(lax.rem(iota, 2), jnp.full((16,), 64, jnp.int32))

        def issue_in(c, s):
            pltpu.async_copy(
                embt_hbm.at[:, pl.ds(c * 128, 128)], in_v.at[s], isems[s]
            )

        def wait_in(s):
            pltpu.make_async_copy(
                embt_hbm.at[:, pl.ds(0, 128)], in_v.at[s], isems[s]
            ).wait()

        def issue_out(c, s):
            pltpu.async_copy(
                out_v.at[s], out_hbm.at[pl.ds(c * 64, 64)], osems[s]
            )

        def wait_out(s):
            pltpu.make_async_copy(
                out_v.at[s], out_hbm.at[pl.ds(0, 64)], osems[s]
            ).wait()

        def transpose_rows(src, dst, d0, nrows):
            psub = lax.shift_right_logical(iota, 1)

            @plsc.parallel_loop(d0, d0 + nrows, step=1, unroll=4)
            def _row(d):
                cidx = cbase + d
                for j in range(8):
                    val = src[d, pl.ds(16 * j, 16)]
                    plsc.store_scatter(
                        dst.at[pl.ds(8 * j, 8)], [psub, cidx], val
                    )

        for s in range(2):
            issue_out(base + s, s)
            issue_in(base + s, s)

        def rstep(i, issue_next):
            for s in range(2):
                c = base + i * 2 + s
                wait_in(s)
                wait_out(s)
                transpose_rows(in_v.at[s], out_v.at[s], 0, 64)
                issue_out(c, s)
                if issue_next:
                    issue_in(c + 2, s)

        lax.fori_loop(0, RPW // 2 - 1, lambda i, cr: (rstep(i, True), cr)[1], 0)
        rstep(RPW // 2 - 1, False)
        wait_out(0)
        wait_out(1)

        @pl.when(wid < NCHUNK - NW * RPW)
        def _():
            c = NW * RPW + wid
            pltpu.sync_copy(embt_hbm.at[:, pl.ds(c * 128, 128)], in_v.at[0])
            transpose_rows(in_v.at[0], out_v.at[0], 0, 64)
            pltpu.sync_copy(out_v.at[0], out_hbm.at[pl.ds(c * 64, 64)])

        @pl.when(wid == NW - 1)
        def _():
            pltpu.sync_copy(tail_hbm, tin_v)

            @plsc.parallel_loop(0, 64, step=1, unroll=4)
            def _trow(d):
                cidx = cbase + d
                for j in range(4):
                    val = tin_v[d, pl.ds(16 * j, 16)]
                    plsc.store_scatter(tout_v, [pidx[j], cidx], val)
            pltpu.sync_copy(
                tout_v, out_hbm.at[pl.ds((V - RTAIL) // 2, RTAIL // 2)]
            )

    return repack


_repack = _repack_build()


def _cbow_pool_build():
    mesh = plsc.VectorSubcoreMesh(core_axis_name="c", subcore_axis_name="s")

    @functools.partial(
        pl.kernel,
        out_type=jax.ShapeDtypeStruct((B, EMB), jnp.float32),
        mesh=mesh,
        compiler_params=pltpu.CompilerParams(use_tc_tiling_on_sc=False),
        scratch_types=[
            pltpu.VMEM((BPW, 128), jnp.int32),
            pltpu.VMEM((BPW, 128), jnp.int32),
            pltpu.VMEM((NBUF, GATHER, EMB), jnp.float32),
            pltpu.VMEM((BPW, EMB), jnp.float32),
            pltpu.SemaphoreType.DMA,
            pltpu.SemaphoreType.DMA,
            pltpu.SemaphoreType.DMA,
            pltpu.SemaphoreType.DMA,
        ],
    )
    def pool(wa_hbm, wb_hbm, emb_hbm, out_hbm, ia_v, ib_v, rows_v, hout_v,
             s0, s1, s2, s3):
        sems = [s0, s1, s2, s3]
        wid = lax.axis_index("s") * NC + lax.axis_index("c")

        pltpu.sync_copy(wa_hbm.at[pl.ds(wid * BPW, BPW)], ia_v)
        pltpu.sync_copy(wb_hbm.at[pl.ds(wid * BPW, BPW)], ib_v)

        def issue(b, s):
            pltpu.async_copy(
                emb_hbm.at[ia_v.at[b]], rows_v.at[s, pl.ds(0, 128)], sems[s]
            )
            pltpu.async_copy(
                emb_hbm.at[ib_v.at[b, pl.ds(56, 72)]],
                rows_v.at[s, pl.ds(128, 72)],
                sems[s],
            )

        for s in range(NBUF):
            issue(s, s)

        def accum(s):
            def body(r10, acc):
                a0, a1, a2, a3 = acc
                for u in range(ROWS_PER_ACC):
                    r = r10 * ROWS_PER_ACC + u
                    a0 = a0 + rows_v[s, r, pl.ds(0, 16)]
                    a1 = a1 + rows_v[s, r, pl.ds(16, 16)]
                    a2 = a2 + rows_v[s, r, pl.ds(32, 16)]
                    a3 = a3 + rows_v[s, r, pl.ds(48, 16)]
                return (a0, a1, a2, a3)
            zeros4 = tuple(jnp.zeros((16,), jnp.float32) for _ in range(4))
            return lax.fori_loop(0, GATHER // ROWS_PER_ACC, body, zeros4)

        def wait(s):
            pltpu.make_async_copy(
                emb_hbm.at[ia_v.at[0]], rows_v.at[s, pl.ds(0, 128)], sems[s]
            ).wait()
            pltpu.make_async_copy(
                emb_hbm.at[ib_v.at[0, pl.ds(56, 72)]],
                rows_v.at[s, pl.ds(128, 72)],
                sems[s],
            ).wait()

        def step_block(i, issue_next):
            for s in range(NBUF):
                wait(s)
                acc = accum(s)
                b = i * NBUF + s
                hout_v[b, pl.ds(0, 16)] = acc[0]
                hout_v[b, pl.ds(16, 16)] = acc[1]
                hout_v[b, pl.ds(32, 16)] = acc[2]
                hout_v[b, pl.ds(48, 16)] = acc[3]
                if issue_next:
                    issue(b + NBUF, s)

        def main_body(i, carry):
            step_block(i, True)
            return carry

        lax.fori_loop(0, STEPS // NBUF - 1, main_body, 0)
        step_block(STEPS // NBUF - 1, False)

        pltpu.sync_copy(hout_v, out_hbm.at[pl.ds(wid * BPW, BPW)])

    return pool


_cbow_pool = _cbow_pool_build()


def _mlp_body(h_ref, w0_ref, b0_ref, w1_ref, b1_ref, wout_ref, bout_ref, o_ref):
    h = h_ref[...]
    t = jnp.tanh(jnp.dot(h, w0_ref[...], preferred_element_type=jnp.float32)
                 + b0_ref[...])
    t = jnp.tanh(jnp.dot(t, w1_ref[...], preferred_element_type=jnp.float32)
                 + b1_ref[...])
    o_ref[...] = (jnp.sum(t * wout_ref[...], axis=1, keepdims=True)
                  + bout_ref[...])


def kernel(words, emb, W0, b0, W1, b1, Wout, bout):
    w32 = words.astype(jnp.int32)
    wa = w32[:, :128]
    wb = w32[:, L - 128:]
    packed = _repack(emb.T, emb[NCHUNK * 128:].T)
    emb_lin = packed.reshape(V, EMB)
    h = _cbow_pool(wa, wb, emb_lin)
    out = pl.pallas_call(
        _mlp_body,
        out_shape=jax.ShapeDtypeStruct((B, 1), jnp.float32),
    )(
        h,
        W0,
        b0.reshape(1, HID),
        W1,
        b1.reshape(1, HID),
        Wout.reshape(1, HID),
        bout.reshape(1, 1),
    )
    return out


# speedup vs baseline: 2.3012x; 1.3837x over previous
import functools

import jax
import jax.numpy as jnp
from jax import lax
from jax.experimental import pallas as pl
from jax.experimental.pallas import tpu as pltpu
from jax.experimental.pallas import tpu_sc as plsc

B = 4096
L = 200
EMB = 64
HID = 128

NC = 2
NS = 16
NW = NC * NS
BPW = B // NW
GATHER = L
STEPS = BPW
NBUF = 4
ROWS_PER_ACC = 10


def _cbow_pool_build():
    mesh = plsc.VectorSubcoreMesh(core_axis_name="c", subcore_axis_name="s")

    @functools.partial(
        pl.kernel,
        out_type=jax.ShapeDtypeStruct((B, EMB), jnp.float32),
        mesh=mesh,
        compiler_params=pltpu.CompilerParams(use_tc_tiling_on_sc=False),
        scratch_types=[
            pltpu.VMEM((BPW, 128), jnp.int32),
            pltpu.VMEM((BPW, 128), jnp.int32),
            pltpu.VMEM((NBUF, GATHER, EMB), jnp.float32),
            pltpu.VMEM((BPW, EMB), jnp.float32),
            pltpu.SemaphoreType.DMA,
            pltpu.SemaphoreType.DMA,
            pltpu.SemaphoreType.DMA,
            pltpu.SemaphoreType.DMA,
        ],
    )
    def pool(wa_hbm, wb_hbm, emb_hbm, out_hbm, ia_v, ib_v, rows_v, hout_v,
             s0, s1, s2, s3):
        sems = [s0, s1, s2, s3]
        wid = lax.axis_index("s") * NC + lax.axis_index("c")

        pltpu.sync_copy(wa_hbm.at[pl.ds(wid * BPW, BPW)], ia_v)
        pltpu.sync_copy(wb_hbm.at[pl.ds(wid * BPW, BPW)], ib_v)

        def issue(b, s):
            pltpu.async_copy(
                emb_hbm.at[ia_v.at[b]], rows_v.at[s, pl.ds(0, 128)], sems[s]
            )
            pltpu.async_copy(
                emb_hbm.at[ib_v.at[b, pl.ds(56, 72)]],
                rows_v.at[s, pl.ds(128, 72)],
                sems[s],
            )

        for s in range(NBUF):
            issue(s, s)

        def accum(s):
            def body(r10, acc):
                a0, a1, a2, a3 = acc
                for u in range(ROWS_PER_ACC):
                    r = r10 * ROWS_PER_ACC + u
                    a0 = a0 + rows_v[s, r, pl.ds(0, 16)]
                    a1 = a1 + rows_v[s, r, pl.ds(16, 16)]
                    a2 = a2 + rows_v[s, r, pl.ds(32, 16)]
                    a3 = a3 + rows_v[s, r, pl.ds(48, 16)]
                return (a0, a1, a2, a3)
            zeros4 = tuple(jnp.zeros((16,), jnp.float32) for _ in range(4))
            return lax.fori_loop(0, GATHER // ROWS_PER_ACC, body, zeros4)

        def wait(s):
            pltpu.make_async_copy(
                emb_hbm.at[ia_v.at[0]], rows_v.at[s, pl.ds(0, 128)], sems[s]
            ).wait()
            pltpu.make_async_copy(
                emb_hbm.at[ib_v.at[0, pl.ds(56, 72)]],
                rows_v.at[s, pl.ds(128, 72)],
                sems[s],
            ).wait()

        def step_block(i, issue_next):
            for s in range(NBUF):
                wait(s)
                acc = accum(s)
                b = i * NBUF + s
                hout_v[b, pl.ds(0, 16)] = acc[0]
                hout_v[b, pl.ds(16, 16)] = acc[1]
                hout_v[b, pl.ds(32, 16)] = acc[2]
                hout_v[b, pl.ds(48, 16)] = acc[3]
                if issue_next:
                    issue(b + NBUF, s)

        def main_body(i, carry):
            step_block(i, True)
            return carry

        lax.fori_loop(0, STEPS // NBUF - 1, main_body, 0)
        step_block(STEPS // NBUF - 1, False)

        pltpu.sync_copy(hout_v, out_hbm.at[pl.ds(wid * BPW, BPW)])

    return pool


_cbow_pool = _cbow_pool_build()


def _mlp_body(h_ref, w0_ref, b0_ref, w1_ref, b1_ref, wout_ref, bout_ref, o_ref):
    h = h_ref[...]
    t = jnp.tanh(jnp.dot(h, w0_ref[...], preferred_element_type=jnp.float32)
                 + b0_ref[...])
    t = jnp.tanh(jnp.dot(t, w1_ref[...], preferred_element_type=jnp.float32)
                 + b1_ref[...])
    o_ref[...] = (jnp.sum(t * wout_ref[...], axis=1, keepdims=True)
                  + bout_ref[...])


def kernel(words, emb, W0, b0, W1, b1, Wout, bout):
    w32 = words.astype(jnp.int32)
    wa = w32[:, :128]
    wb = w32[:, L - 128:]
    h = _cbow_pool(wa, wb, emb)
    out = pl.pallas_call(
        _mlp_body,
        out_shape=jax.ShapeDtypeStruct((B, 1), jnp.float32),
    )(
        h,
        W0,
        b0.reshape(1, HID),
        W1,
        b1.reshape(1, HID),
        Wout.reshape(1, HID),
        bout.reshape(1, 1),
    )
    return out
